# baseline (speedup 1.0000x reference)
import functools

import jax
import jax.numpy as jnp
from jax import lax
from jax.experimental import pallas as pl
from jax.experimental.pallas import tpu as pltpu
from jax.experimental.pallas import tpu_sc as plsc

E = 8
TOPK = 2
DIM = 2048
HIDDEN = 2048

BM = 128
RBM = 512
NWORKERS = 32
L = 16


def _router_body(x_ref, wr_ref, sel_ref, wts_ref, rank_ref, cnt_ref,
                 base_ref, cnt_scratch):
    i = pl.program_id(0)

    @pl.when(i == 0)
    def _():
        cnt_scratch[...] = jnp.zeros_like(cnt_scratch)

    xb = x_ref[...]
    wr = wr_ref[...]
    logits = jax.lax.dot_general(
        xb, wr, (((1,), (1,)), ((), ())),
        preferred_element_type=jnp.float32,
        precision=jax.lax.Precision.DEFAULT)
    iota = jax.lax.broadcasted_iota(jnp.int32, logits.shape, 1)
    m1 = jnp.max(logits, axis=1, keepdims=True)
    i1 = jnp.min(jnp.where(logits == m1, iota, E), axis=1, keepdims=True)
    masked = jnp.where(iota == i1, -jnp.inf, logits)
    m2 = jnp.max(masked, axis=1, keepdims=True)
    i2 = jnp.min(jnp.where(masked == m2, iota, E), axis=1, keepdims=True)
    w0 = 1.0 / (1.0 + jnp.exp(m2 - m1))
    sel_ref[...] = jnp.concatenate([i1, i2], axis=1)
    wts_ref[...] = jnp.concatenate([w0, 1.0 - w0], axis=1)

    oh0 = (iota == i1).astype(jnp.float32)
    oh1 = (iota == i2).astype(jnp.float32)
    r_iota = jax.lax.broadcasted_iota(jnp.int32, (RBM, RBM), 0)
    c_iota = jax.lax.broadcasted_iota(jnp.int32, (RBM, RBM), 1)
    strict_tril = (r_iota > c_iota).astype(jnp.float32)
    excl0 = jax.lax.dot_general(
        strict_tril, oh0, (((1,), (0,)), ((), ())),
        preferred_element_type=jnp.float32)
    excl1 = jax.lax.dot_general(
        strict_tril, oh1, (((1,), (0,)), ((), ())),
        preferred_element_type=jnp.float32)
    tot0 = jnp.sum(oh0, axis=0, keepdims=True)
    tot1 = jnp.sum(oh1, axis=0, keepdims=True)
    cnt = cnt_scratch[...]
    rank0 = jnp.sum(oh0 * (excl0 + cnt), axis=1, keepdims=True)
    rank1 = jnp.sum(oh1 * (excl1 + cnt + tot0), axis=1, keepdims=True)
    rank_ref[...] = jnp.concatenate([rank0, rank1], axis=1).astype(jnp.int32)
    new_cnt = cnt + tot0 + tot1
    cnt_scratch[...] = new_cnt
    cnt_ref[...] = new_cnt.astype(jnp.int32)
    e_r = jax.lax.broadcasted_iota(jnp.int32, (E, E), 0)
    e_c = jax.lax.broadcasted_iota(jnp.int32, (E, E), 1)
    strict = (e_r < e_c).astype(jnp.float32)
    base_ref[...] = jax.lax.dot_general(
        new_cnt, strict, (((1,), (0,)), ((), ())),
        preferred_element_type=jnp.float32,
        precision=jax.lax.Precision.HIGHEST).astype(jnp.int32)


def _router(xt, w_router):
    T = xt.shape[0]
    return pl.pallas_call(
        _router_body,
        grid=(T // RBM,),
        in_specs=[
            pl.BlockSpec((RBM, DIM), lambda i: (i, 0)),
            pl.BlockSpec((E, DIM), lambda i: (0, 0)),
        ],
        out_specs=[
            pl.BlockSpec((RBM, TOPK), lambda i: (i, 0)),
            pl.BlockSpec((RBM, TOPK), lambda i: (i, 0)),
            pl.BlockSpec((RBM, TOPK), lambda i: (i, 0)),
            pl.BlockSpec((1, E), lambda i: (0, 0)),
            pl.BlockSpec((1, E), lambda i: (0, 0)),
        ],
        out_shape=[
            jax.ShapeDtypeStruct((T, TOPK), jnp.int32),
            jax.ShapeDtypeStruct((T, TOPK), jnp.float32),
            jax.ShapeDtypeStruct((T, TOPK), jnp.int32),
            jax.ShapeDtypeStruct((1, E), jnp.int32),
            jax.ShapeDtypeStruct((1, E), jnp.int32),
        ],
        scratch_shapes=[pltpu.VMEM((1, E), jnp.float32)],
    )(xt, w_router)


def _make_dispatch(T):
    t_per_w = T // NWORKERS
    n_chunks = t_per_w // L
    mesh = plsc.VectorSubcoreMesh(core_axis_name="c", subcore_axis_name="s")

    @functools.partial(
        pl.kernel, mesh=mesh,
        out_type=[
            jax.ShapeDtypeStruct((T * TOPK, DIM), jnp.float32),
            jax.ShapeDtypeStruct((T,), jnp.int32),
            jax.ShapeDtypeStruct((T,), jnp.int32),
        ],
        scratch_types=[
            pltpu.VMEM((L,), jnp.int32),
            pltpu.VMEM((t_per_w,), jnp.int32),
            pltpu.VMEM((t_per_w,), jnp.int32),
            pltpu.VMEM((t_per_w,), jnp.int32),
            pltpu.VMEM((t_per_w,), jnp.int32),
            pltpu.VMEM((t_per_w,), jnp.int32),
            pltpu.VMEM((t_per_w,), jnp.int32),
            pltpu.VMEM((L, DIM), jnp.float32),
            pltpu.VMEM((L, DIM), jnp.float32),
            pltpu.SemaphoreType.DMA,
            pltpu.SemaphoreType.DMA,
            pltpu.SemaphoreType.DMA,
            pltpu.SemaphoreType.DMA,
        ],
        compiler_params=pltpu.CompilerParams(needs_layout_passes=False),
    )
    def dispatch(xt_hbm, sel0_hbm, sel1_hbm, rank0_hbm, rank1_hbm, base_hbm,
                 xs_hbm, pos0_hbm, pos1_hbm,
                 base_v, sel0_v, sel1_v, rank0_v, rank1_v,
                 pos0_v, pos1_v, buf_a, buf_b, lsem_a, lsem_b,
                 ssem_a, ssem_b, ):
        wid = lax.axis_index("s") * 2 + lax.axis_index("c")
        tbase = wid * t_per_w
        pltpu.sync_copy(base_hbm, base_v)
        pltpu.sync_copy(sel0_hbm.at[pl.ds(tbase, t_per_w)], sel0_v)
        pltpu.sync_copy(sel1_hbm.at[pl.ds(tbase, t_per_w)], sel1_v)
        pltpu.sync_copy(rank0_hbm.at[pl.ds(tbase, t_per_w)], rank0_v)
        pltpu.sync_copy(rank1_hbm.at[pl.ds(tbase, t_per_w)], rank1_v)
        bufs = (buf_a, buf_b)
        lsems = (lsem_a, lsem_b)
        ssems = (ssem_a, ssem_b)
        loads = [None] * n_chunks
        scats = [None] * n_chunks
        loads[0] = pltpu.async_copy(
            xt_hbm.at[pl.ds(tbase, L)], bufs[0], lsems[0])
        for c in range(n_chunks):
            p = c % 2
            loads[c].wait()
            if c + 1 < n_chunks:
                if c >= 1:
                    scats[c - 1][0].wait()
                    scats[c - 1][1].wait()
                loads[c + 1] = pltpu.async_copy(
                    xt_hbm.at[pl.ds(tbase + (c + 1) * L, L)],
                    bufs[(c + 1) % 2], lsems[(c + 1) % 2])
            v0 = sel0_v[pl.ds(c * L, L)]
            v1 = sel1_v[pl.ds(c * L, L)]
            p0 = plsc.load_gather(base_v, [v0]) + rank0_v[pl.ds(c * L, L)]
            p1 = plsc.load_gather(base_v, [v1]) + rank1_v[pl.ds(c * L, L)]
            pos0_v[pl.ds(c * L, L)] = p0
            pos1_v[pl.ds(c * L, L)] = p1
            scats[c] = (
                pltpu.async_copy(bufs[p], xs_hbm.at[p0], ssems[p]),
                pltpu.async_copy(bufs[p], xs_hbm.at[p1], ssems[p]))
        scats[n_chunks - 2][0].wait()
        scats[n_chunks - 2][1].wait()
        scats[n_chunks - 1][0].wait()
        scats[n_chunks - 1][1].wait()
        pltpu.sync_copy(pos0_v, pos0_hbm.at[pl.ds(tbase, t_per_w)])
        pltpu.sync_copy(pos1_v, pos1_hbm.at[pl.ds(tbase, t_per_w)])

    return dispatch


def _make_gather(T):
    t_per_w = T // NWORKERS
    n_chunks = t_per_w // L
    nbuf = 3
    mesh = plsc.VectorSubcoreMesh(core_axis_name="c", subcore_axis_name="s")

    @functools.partial(
        pl.kernel, mesh=mesh,
        out_type=[
            jax.ShapeDtypeStruct((T, DIM), jnp.float32),
            jax.ShapeDtypeStruct((T, DIM), jnp.float32),
        ],
        scratch_types=[
            pltpu.VMEM((t_per_w,), jnp.int32),
            pltpu.VMEM((t_per_w,), jnp.int32),
            pltpu.VMEM((L, DIM), jnp.float32),
            pltpu.VMEM((L, DIM), jnp.float32),
            pltpu.VMEM((L, DIM), jnp.float32),
            pltpu.SemaphoreType.DMA,
            pltpu.SemaphoreType.DMA,
            pltpu.SemaphoreType.DMA,
        ],
        compiler_params=pltpu.CompilerParams(needs_layout_passes=False),
    )
    def gather(ys_hbm, pos0_hbm, pos1_hbm, g0_hbm, g1_hbm,
               pos0_v, pos1_v, bufa, bufb, bufc, sema, semb, semc):
        wid = lax.axis_index("s") * 2 + lax.axis_index("c")
        tbase = wid * t_per_w
        pltpu.sync_copy(pos0_hbm.at[pl.ds(tbase, t_per_w)], pos0_v)
        pltpu.sync_copy(pos1_hbm.at[pl.ds(tbase, t_per_w)], pos1_v)
        bufs = (bufa, bufb, bufc)
        sems = (sema, semb, semc)
        pos_vs = (pos0_v, pos1_v)
        g_hbms = (g0_hbm, g1_hbm)
        n_tr = 2 * n_chunks

        def start(i):
            c, col = divmod(i, 2)
            q = pos_vs[col][pl.ds(c * L, L)]
            return pltpu.async_copy(ys_hbm.at[q], bufs[i % nbuf],
                                    sems[i % nbuf])

        pend = [None] * n_tr
        for i in range(min(nbuf, n_tr)):
            pend[i] = start(i)
        for i in range(n_tr):
            c, col = divmod(i, 2)
            pend[i].wait()
            pltpu.sync_copy(bufs[i % nbuf],
                            g_hbms[col].at[pl.ds(tbase + c * L, L)])
            if i + nbuf < n_tr:
                pend[i + nbuf] = start(i + nbuf)

    return gather


def _gffn_body(meta_ref, x_ref, w1_ref, w3_ref, w2_ref, out_ref):
    i = pl.program_id(0)
    first = meta_ref[2, i]
    lo = meta_ref[3, i]
    hi = meta_ref[4, i]
    m = meta_ref[1, i]

    xb = x_ref[...].astype(jnp.bfloat16)
    a = jax.lax.dot_general(
        xb, w1_ref[0], (((1,), (0,)), ((), ())),
        preferred_element_type=jnp.float32)
    b = jax.lax.dot_general(
        xb, w3_ref[0], (((1,), (0,)), ((), ())),
        preferred_element_type=jnp.float32)
    h = (a * (1.0 / (1.0 + jnp.exp(-a))) * b).astype(jnp.bfloat16)
    y = jax.lax.dot_general(
        h, w2_ref[0], (((1,), (0,)), ((), ())),
        preferred_element_type=jnp.float32)

    rows = m * BM + jax.lax.broadcasted_iota(jnp.int32, (BM, 1), 0)
    y = jnp.where((rows >= lo) & (rows < hi), y, 0.0)

    @pl.when(first == 1)
    def _():
        out_ref[...] = y

    @pl.when(first == 0)
    def _():
        out_ref[...] += y


def _gffn(xs, w1, w3, w2, meta, n_items):
    R = xs.shape[0]
    grid_spec = pltpu.PrefetchScalarGridSpec(
        num_scalar_prefetch=1,
        grid=(n_items,),
        in_specs=[
            pl.BlockSpec((BM, DIM), lambda i, meta: (meta[1, i], 0)),
            pl.BlockSpec((1, DIM, HIDDEN), lambda i, meta: (meta[0, i], 0, 0)),
            pl.BlockSpec((1, DIM, HIDDEN), lambda i, meta: (meta[0, i], 0, 0)),
            pl.BlockSpec((1, HIDDEN, DIM), lambda i, meta: (meta[0, i], 0, 0)),
        ],
        out_specs=pl.BlockSpec((BM, DIM), lambda i, meta: (meta[1, i], 0)),
    )
    return pl.pallas_call(
        _gffn_body,
        grid_spec=grid_spec,
        out_shape=jax.ShapeDtypeStruct((R, DIM), jnp.float32),
    )(meta, xs, w1, w3, w2)


def _expert_meta(counts, n_rows, n_items):
    ends = jnp.cumsum(counts)
    starts = ends - counts
    f = starts // BM
    l = (ends - 1) // BM
    tiles = jnp.where(counts > 0, l - f + 1, 0)
    c_incl = jnp.cumsum(tiles)
    c_excl = c_incl - tiles
    n_real = c_incl[-1]
    i = jnp.arange(n_items, dtype=jnp.int32)
    e_of = jnp.sum(c_incl[None, :] <= i[:, None], axis=1)
    e_of = jnp.clip(e_of, 0, counts.shape[0] - 1).astype(jnp.int32)
    m_of = (f[e_of] + (i - c_excl[e_of])).astype(jnp.int32)
    valid = i < n_real
    last_m = (n_rows // BM) - 1
    m_of = jnp.where(valid, m_of, last_m)
    lo = jnp.where(valid, jnp.maximum(starts[e_of], m_of * BM), n_rows)
    hi = jnp.where(valid, jnp.minimum(ends[e_of], (m_of + 1) * BM), n_rows)
    first = jnp.concatenate(
        [jnp.ones((1,), jnp.int32),
         (m_of[1:] != m_of[:-1]).astype(jnp.int32)])
    first = jnp.where(valid, first, 0)
    return jnp.stack([e_of, m_of, first,
                      lo.astype(jnp.int32), hi.astype(jnp.int32)]).astype(jnp.int32)


def _combine_body(sh_ref, g0_ref, g1_ref, w0_ref, w1_ref, out_ref):
    out_ref[...] = (sh_ref[...] + w0_ref[...] * g0_ref[...]
                    + w1_ref[...] * g1_ref[...])


def _combine(shared, g0, g1, w0, w1):
    T = shared.shape[0]
    CB = 512
    return pl.pallas_call(
        _combine_body,
        grid=(T // CB,),
        in_specs=[
            pl.BlockSpec((CB, DIM), lambda i: (i, 0)),
            pl.BlockSpec((CB, DIM), lambda i: (i, 0)),
            pl.BlockSpec((CB, DIM), lambda i: (i, 0)),
            pl.BlockSpec((CB, 1), lambda i: (i, 0)),
            pl.BlockSpec((CB, 1), lambda i: (i, 0)),
        ],
        out_specs=pl.BlockSpec((CB, DIM), lambda i: (i, 0)),
        out_shape=jax.ShapeDtypeStruct((T, DIM), jnp.float32),
    )(shared, g0, g1, w0, w1)


def kernel(x, w_router, w1, w2, w3, sw1, sw2, sw3):
    bs, slen, dim = x.shape
    T = bs * slen
    R = T * TOPK
    xt = x.reshape(T, dim)

    w1b = w1.astype(jnp.bfloat16)
    w2b = w2.astype(jnp.bfloat16)
    w3b = w3.astype(jnp.bfloat16)
    xtb = xt.astype(jnp.bfloat16)

    sel, wts, rank, counts, base = _router(xt, w_router)

    xs, pos0, pos1 = _make_dispatch(T)(
        xt, sel[:, 0], sel[:, 1], rank[:, 0], rank[:, 1],
        jnp.pad(base[0], (0, L - E)))

    n_items = R // BM + E - 1
    meta = _expert_meta(counts[0], R, n_items)
    ys = _gffn(xs, w1b, w3b, w2b, meta, n_items)

    n_tiles_s = T // BM
    ms = jnp.arange(n_tiles_s, dtype=jnp.int32)
    meta_s = jnp.stack([
        jnp.zeros(n_tiles_s, jnp.int32), ms, jnp.ones(n_tiles_s, jnp.int32),
        ms * BM, (ms + 1) * BM]).astype(jnp.int32)
    shared = _gffn(xtb, sw1.astype(jnp.bfloat16)[None],
                   sw3.astype(jnp.bfloat16)[None],
                   sw2.astype(jnp.bfloat16)[None], meta_s, n_tiles_s)

    g0, g1 = _make_gather(T)(ys, pos0, pos1)
    out = _combine(shared, g0, g1, wts[:, 0:1], wts[:, 1:2])
    return out.reshape(bs, slen, dim)


# speedup vs baseline: 1.1238x; 1.1238x over previous
import functools

import jax
import jax.numpy as jnp
from jax import lax
from jax.experimental import pallas as pl
from jax.experimental.pallas import tpu as pltpu
from jax.experimental.pallas import tpu_sc as plsc

E = 8
TOPK = 2
DIM = 2048
HIDDEN = 2048

BM = 128
RBM = 512
NWORKERS = 32
L = 16


def _router_body(x_ref, wr_ref, sel_ref, wts_ref, rank_ref, cnt_ref,
                 base_ref, cnt_scratch):
    i = pl.program_id(0)

    @pl.when(i == 0)
    def _():
        cnt_scratch[...] = jnp.zeros_like(cnt_scratch)

    xb = x_ref[...]
    wr = wr_ref[...]
    logits = jax.lax.dot_general(
        xb, wr, (((1,), (1,)), ((), ())),
        preferred_element_type=jnp.float32,
        precision=jax.lax.Precision.DEFAULT)
    iota = jax.lax.broadcasted_iota(jnp.int32, logits.shape, 1)
    m1 = jnp.max(logits, axis=1, keepdims=True)
    i1 = jnp.min(jnp.where(logits == m1, iota, E), axis=1, keepdims=True)
    masked = jnp.where(iota == i1, -jnp.inf, logits)
    m2 = jnp.max(masked, axis=1, keepdims=True)
    i2 = jnp.min(jnp.where(masked == m2, iota, E), axis=1, keepdims=True)
    w0 = 1.0 / (1.0 + jnp.exp(m2 - m1))
    sel_ref[...] = jnp.concatenate([i1, i2], axis=1)
    wts_ref[...] = jnp.concatenate([w0, 1.0 - w0], axis=1)

    oh0 = (iota == i1).astype(jnp.float32)
    oh1 = (iota == i2).astype(jnp.float32)
    r_iota = jax.lax.broadcasted_iota(jnp.int32, (RBM, RBM), 0)
    c_iota = jax.lax.broadcasted_iota(jnp.int32, (RBM, RBM), 1)
    strict_tril = (r_iota > c_iota).astype(jnp.float32)
    excl0 = jax.lax.dot_general(
        strict_tril, oh0, (((1,), (0,)), ((), ())),
        preferred_element_type=jnp.float32)
    excl1 = jax.lax.dot_general(
        strict_tril, oh1, (((1,), (0,)), ((), ())),
        preferred_element_type=jnp.float32)
    tot0 = jnp.sum(oh0, axis=0, keepdims=True)
    tot1 = jnp.sum(oh1, axis=0, keepdims=True)
    cnt = cnt_scratch[...]
    rank0 = jnp.sum(oh0 * (excl0 + cnt), axis=1, keepdims=True)
    rank1 = jnp.sum(oh1 * (excl1 + cnt + tot0), axis=1, keepdims=True)
    rank_ref[...] = jnp.concatenate([rank0, rank1], axis=1).astype(jnp.int32)
    new_cnt = cnt + tot0 + tot1
    cnt_scratch[...] = new_cnt
    cnt_ref[...] = new_cnt.astype(jnp.int32)
    e_r = jax.lax.broadcasted_iota(jnp.int32, (E, E), 0)
    e_c = jax.lax.broadcasted_iota(jnp.int32, (E, E), 1)
    strict = (e_r < e_c).astype(jnp.float32)
    base_ref[...] = jax.lax.dot_general(
        new_cnt, strict, (((1,), (0,)), ((), ())),
        preferred_element_type=jnp.float32,
        precision=jax.lax.Precision.HIGHEST).astype(jnp.int32)


def _router(xt, w_router):
    T = xt.shape[0]
    return pl.pallas_call(
        _router_body,
        grid=(T // RBM,),
        in_specs=[
            pl.BlockSpec((RBM, DIM), lambda i: (i, 0)),
            pl.BlockSpec((E, DIM), lambda i: (0, 0)),
        ],
        out_specs=[
            pl.BlockSpec((RBM, TOPK), lambda i: (i, 0)),
            pl.BlockSpec((RBM, TOPK), lambda i: (i, 0)),
            pl.BlockSpec((RBM, TOPK), lambda i: (i, 0)),
            pl.BlockSpec((1, E), lambda i: (0, 0)),
            pl.BlockSpec((1, E), lambda i: (0, 0)),
        ],
        out_shape=[
            jax.ShapeDtypeStruct((T, TOPK), jnp.int32),
            jax.ShapeDtypeStruct((T, TOPK), jnp.float32),
            jax.ShapeDtypeStruct((T, TOPK), jnp.int32),
            jax.ShapeDtypeStruct((1, E), jnp.int32),
            jax.ShapeDtypeStruct((1, E), jnp.int32),
        ],
        scratch_shapes=[pltpu.VMEM((1, E), jnp.float32)],
    )(xt, w_router)


def _make_dispatch(T):
    t_per_w = T // NWORKERS
    n_chunks = t_per_w // L
    mesh = plsc.VectorSubcoreMesh(core_axis_name="c", subcore_axis_name="s")

    @functools.partial(
        pl.kernel, mesh=mesh,
        out_type=[
            jax.ShapeDtypeStruct((T * TOPK, DIM), jnp.float32),
            jax.ShapeDtypeStruct((T,), jnp.int32),
            jax.ShapeDtypeStruct((T,), jnp.int32),
        ],
        scratch_types=[
            pltpu.VMEM((L,), jnp.int32),
            pltpu.VMEM((t_per_w,), jnp.int32),
            pltpu.VMEM((t_per_w,), jnp.int32),
            pltpu.VMEM((t_per_w,), jnp.int32),
            pltpu.VMEM((t_per_w,), jnp.int32),
            pltpu.VMEM((t_per_w,), jnp.int32),
            pltpu.VMEM((t_per_w,), jnp.int32),
            pltpu.VMEM((L, DIM), jnp.float32),
            pltpu.VMEM((L, DIM), jnp.float32),
            pltpu.SemaphoreType.DMA,
            pltpu.SemaphoreType.DMA,
            pltpu.SemaphoreType.DMA,
            pltpu.SemaphoreType.DMA,
        ],
        compiler_params=pltpu.CompilerParams(needs_layout_passes=False),
    )
    def dispatch(xt_hbm, sel0_hbm, sel1_hbm, rank0_hbm, rank1_hbm, base_hbm,
                 xs_hbm, pos0_hbm, pos1_hbm,
                 base_v, sel0_v, sel1_v, rank0_v, rank1_v,
                 pos0_v, pos1_v, buf_a, buf_b, lsem_a, lsem_b,
                 ssem_a, ssem_b, ):
        wid = lax.axis_index("s") * 2 + lax.axis_index("c")
        tbase = wid * t_per_w
        pltpu.sync_copy(base_hbm, base_v)
        pltpu.sync_copy(sel0_hbm.at[pl.ds(tbase, t_per_w)], sel0_v)
        pltpu.sync_copy(sel1_hbm.at[pl.ds(tbase, t_per_w)], sel1_v)
        pltpu.sync_copy(rank0_hbm.at[pl.ds(tbase, t_per_w)], rank0_v)
        pltpu.sync_copy(rank1_hbm.at[pl.ds(tbase, t_per_w)], rank1_v)
        bufs = (buf_a, buf_b)
        lsems = (lsem_a, lsem_b)
        ssems = (ssem_a, ssem_b)
        loads = [None] * n_chunks
        scats = [None] * n_chunks
        loads[0] = pltpu.async_copy(
            xt_hbm.at[pl.ds(tbase, L)], bufs[0], lsems[0])
        for c in range(n_chunks):
            p = c % 2
            loads[c].wait()
            if c + 1 < n_chunks:
                if c >= 1:
                    scats[c - 1][0].wait()
                    scats[c - 1][1].wait()
                loads[c + 1] = pltpu.async_copy(
                    xt_hbm.at[pl.ds(tbase + (c + 1) * L, L)],
                    bufs[(c + 1) % 2], lsems[(c + 1) % 2])
            v0 = sel0_v[pl.ds(c * L, L)]
            v1 = sel1_v[pl.ds(c * L, L)]
            p0 = plsc.load_gather(base_v, [v0]) + rank0_v[pl.ds(c * L, L)]
            p1 = plsc.load_gather(base_v, [v1]) + rank1_v[pl.ds(c * L, L)]
            pos0_v[pl.ds(c * L, L)] = p0
            pos1_v[pl.ds(c * L, L)] = p1
            scats[c] = (
                pltpu.async_copy(bufs[p], xs_hbm.at[p0], ssems[p]),
                pltpu.async_copy(bufs[p], xs_hbm.at[p1], ssems[p]))
        scats[n_chunks - 2][0].wait()
        scats[n_chunks - 2][1].wait()
        scats[n_chunks - 1][0].wait()
        scats[n_chunks - 1][1].wait()
        pltpu.sync_copy(pos0_v, pos0_hbm.at[pl.ds(tbase, t_per_w)])
        pltpu.sync_copy(pos1_v, pos1_hbm.at[pl.ds(tbase, t_per_w)])

    return dispatch


def _make_gather(T):
    t_per_w = T // NWORKERS
    n_chunks = t_per_w // L
    nbuf = 3
    mesh = plsc.VectorSubcoreMesh(core_axis_name="c", subcore_axis_name="s")

    @functools.partial(
        pl.kernel, mesh=mesh,
        out_type=[
            jax.ShapeDtypeStruct((T, DIM), jnp.float32),
            jax.ShapeDtypeStruct((T, DIM), jnp.float32),
        ],
        scratch_types=[
            pltpu.VMEM((t_per_w,), jnp.int32),
            pltpu.VMEM((t_per_w,), jnp.int32),
            pltpu.VMEM((L, DIM), jnp.float32),
            pltpu.VMEM((L, DIM), jnp.float32),
            pltpu.VMEM((L, DIM), jnp.float32),
            pltpu.SemaphoreType.DMA,
            pltpu.SemaphoreType.DMA,
            pltpu.SemaphoreType.DMA,
        ],
        compiler_params=pltpu.CompilerParams(needs_layout_passes=False),
    )
    def gather(ys_hbm, pos0_hbm, pos1_hbm, g0_hbm, g1_hbm,
               pos0_v, pos1_v, bufa, bufb, bufc, sema, semb, semc):
        wid = lax.axis_index("s") * 2 + lax.axis_index("c")
        tbase = wid * t_per_w
        pltpu.sync_copy(pos0_hbm.at[pl.ds(tbase, t_per_w)], pos0_v)
        pltpu.sync_copy(pos1_hbm.at[pl.ds(tbase, t_per_w)], pos1_v)
        bufs = (bufa, bufb, bufc)
        sems = (sema, semb, semc)
        pos_vs = (pos0_v, pos1_v)
        g_hbms = (g0_hbm, g1_hbm)
        n_tr = 2 * n_chunks

        def start(i):
            c, col = divmod(i, 2)
            q = pos_vs[col][pl.ds(c * L, L)]
            return pltpu.async_copy(ys_hbm.at[q], bufs[i % nbuf],
                                    sems[i % nbuf])

        pend = [None] * n_tr
        for i in range(min(nbuf, n_tr)):
            pend[i] = start(i)
        for i in range(n_tr):
            c, col = divmod(i, 2)
            pend[i].wait()
            pltpu.sync_copy(bufs[i % nbuf],
                            g_hbms[col].at[pl.ds(tbase + c * L, L)])
            if i + nbuf < n_tr:
                pend[i + nbuf] = start(i + nbuf)

    return gather


HID2 = HIDDEN // 2


def _gffn_half_body(has_prev, meta_ref, *refs):
    if has_prev:
        x_ref, w1_ref, w3_ref, w2_ref, prev_ref, out_ref = refs
    else:
        x_ref, w1_ref, w3_ref, w2_ref, out_ref = refs
    i = pl.program_id(0)
    first = meta_ref[2, i]
    lo = meta_ref[3, i]
    hi = meta_ref[4, i]
    m = meta_ref[1, i]

    xb = x_ref[...].astype(jnp.bfloat16)
    a = jax.lax.dot_general(
        xb, w1_ref[0].astype(jnp.bfloat16), (((1,), (0,)), ((), ())),
        preferred_element_type=jnp.float32)
    b = jax.lax.dot_general(
        xb, w3_ref[0].astype(jnp.bfloat16), (((1,), (0,)), ((), ())),
        preferred_element_type=jnp.float32)
    h = (a * (1.0 / (1.0 + jnp.exp(-a))) * b).astype(jnp.bfloat16)
    y = jax.lax.dot_general(
        h, w2_ref[0].astype(jnp.bfloat16), (((1,), (0,)), ((), ())),
        preferred_element_type=jnp.float32)

    rows = m * BM + jax.lax.broadcasted_iota(jnp.int32, (BM, 1), 0)
    y = jnp.where((rows >= lo) & (rows < hi), y, 0.0)
    if has_prev:
        @pl.when(first == 1)
        def _():
            out_ref[...] = prev_ref[...] + y

        @pl.when(first == 0)
        def _():
            out_ref[...] += y
    else:
        @pl.when(first == 1)
        def _():
            out_ref[...] = y

        @pl.when(first == 0)
        def _():
            out_ref[...] += y


def _gffn_half(xs, w1, w3, w2, meta, n_items, nh, prev=None):
    R = xs.shape[0]
    in_specs = [
        pl.BlockSpec((BM, DIM), lambda i, meta: (meta[1, i], 0)),
        pl.BlockSpec((1, DIM, HID2), lambda i, meta: (meta[0, i], 0, nh)),
        pl.BlockSpec((1, DIM, HID2), lambda i, meta: (meta[0, i], 0, nh)),
        pl.BlockSpec((1, HID2, DIM), lambda i, meta: (meta[0, i], nh, 0)),
    ]
    args = [meta, xs, w1, w3, w2]
    kwargs = {}
    if prev is not None:
        in_specs.append(pl.BlockSpec((BM, DIM), lambda i, meta: (meta[1, i], 0)))
        args.append(prev)
        kwargs["input_output_aliases"] = {5: 0}
    grid_spec = pltpu.PrefetchScalarGridSpec(
        num_scalar_prefetch=1,
        grid=(n_items,),
        in_specs=in_specs,
        out_specs=pl.BlockSpec((BM, DIM), lambda i, meta: (meta[1, i], 0)),
    )
    return pl.pallas_call(
        functools.partial(_gffn_half_body, prev is not None),
        grid_spec=grid_spec,
        out_shape=jax.ShapeDtypeStruct((R, DIM), jnp.float32),
        **kwargs,
    )(*args)


def _gffn(xs, w1, w3, w2, meta, n_items):
    part = _gffn_half(xs, w1, w3, w2, meta, n_items, 0)
    return _gffn_half(xs, w1, w3, w2, meta, n_items, 1, prev=part)


def _expert_meta(counts, n_rows, n_items):
    ends = jnp.cumsum(counts)
    starts = ends - counts
    f = starts // BM
    l = (ends - 1) // BM
    tiles = jnp.where(counts > 0, l - f + 1, 0)
    c_incl = jnp.cumsum(tiles)
    c_excl = c_incl - tiles
    n_real = c_incl[-1]
    i = jnp.arange(n_items, dtype=jnp.int32)
    e_of = jnp.sum(c_incl[None, :] <= i[:, None], axis=1)
    e_of = jnp.clip(e_of, 0, counts.shape[0] - 1).astype(jnp.int32)
    m_of = (f[e_of] + (i - c_excl[e_of])).astype(jnp.int32)
    valid = i < n_real
    last_m = (n_rows // BM) - 1
    m_of = jnp.where(valid, m_of, last_m)
    lo = jnp.where(valid, jnp.maximum(starts[e_of], m_of * BM), n_rows)
    hi = jnp.where(valid, jnp.minimum(ends[e_of], (m_of + 1) * BM), n_rows)
    first = jnp.concatenate(
        [jnp.ones((1,), jnp.int32),
         (m_of[1:] != m_of[:-1]).astype(jnp.int32)])
    first = jnp.where(valid, first, 0)
    return jnp.stack([e_of, m_of, first,
                      lo.astype(jnp.int32), hi.astype(jnp.int32)]).astype(jnp.int32)


def _combine_body(sh_ref, g0_ref, g1_ref, w0_ref, w1_ref, out_ref):
    out_ref[...] = (sh_ref[...] + w0_ref[...] * g0_ref[...]
                    + w1_ref[...] * g1_ref[...])


def _combine(shared, g0, g1, w0, w1):
    T = shared.shape[0]
    CB = 512
    return pl.pallas_call(
        _combine_body,
        grid=(T // CB,),
        in_specs=[
            pl.BlockSpec((CB, DIM), lambda i: (i, 0)),
            pl.BlockSpec((CB, DIM), lambda i: (i, 0)),
            pl.BlockSpec((CB, DIM), lambda i: (i, 0)),
            pl.BlockSpec((CB, 1), lambda i: (i, 0)),
            pl.BlockSpec((CB, 1), lambda i: (i, 0)),
        ],
        out_specs=pl.BlockSpec((CB, DIM), lambda i: (i, 0)),
        out_shape=jax.ShapeDtypeStruct((T, DIM), jnp.float32),
    )(shared, g0, g1, w0, w1)


def kernel(x, w_router, w1, w2, w3, sw1, sw2, sw3):
    bs, slen, dim = x.shape
    T = bs * slen
    R = T * TOPK
    xt = x.reshape(T, dim)

    xtb = xt.astype(jnp.bfloat16)

    sel, wts, rank, counts, base = _router(xt, w_router)

    xs, pos0, pos1 = _make_dispatch(T)(
        xt, sel[:, 0], sel[:, 1], rank[:, 0], rank[:, 1],
        jnp.pad(base[0], (0, L - E)))

    n_items = R // BM + E - 1
    meta = _expert_meta(counts[0], R, n_items)
    ys = _gffn(xs, w1, w3, w2, meta, n_items)

    n_tiles_s = T // BM
    ms = jnp.arange(n_tiles_s, dtype=jnp.int32)
    meta_s = jnp.stack([
        jnp.zeros(n_tiles_s, jnp.int32), ms, jnp.ones(n_tiles_s, jnp.int32),
        ms * BM, (ms + 1) * BM]).astype(jnp.int32)
    shared = _gffn(xtb, sw1[None], sw3[None], sw2[None], meta_s, n_tiles_s)

    g0, g1 = _make_gather(T)(ys, pos0, pos1)
    out = _combine(shared, g0, g1, wts[:, 0:1], wts[:, 1:2])
    return out.reshape(bs, slen, dim)
